# Initial kernel scaffold; baseline (speedup 1.0000x reference)
import jax
import jax.numpy as jnp
from jax.experimental import pallas as pl


def kernel(segmentation_results, lidar_raw, Tr_velo_to_cam_0, Tr_velo_to_cam_1, Tr_velo_to_cam_2, Tr_velo_to_cam_3, Tr_velo_to_cam_4, P0, P1, P2, P3, P4, R0_rect):
    raise NotImplementedError("write your pallas kernel here")


# speedup vs baseline: 5.9182x; 5.9182x over previous
import functools

import jax
import jax.numpy as jnp
from jax import lax
from jax.experimental import pallas as pl
from jax.experimental.pallas import tpu as pltpu
from jax.experimental.pallas import tpu_sc as plsc

H = 384
W = 1024
HW = H * W
NPTS = 200000

NW = 32
CHUNK = 1024
CPW = 7
NP = NW * CPW * CHUNK
NCH = NP // CHUNK

GROUPS = [list(range(11)), [18], [13, 14, 15, 16], [11], [12], [17]]

BH = 8
BN = 8192


def _softmax_body(x_ref, o_ref, e_scr):
    for k in range(19):
        e_scr[k] = jnp.exp(x_ref[0, k])
    sums = []
    for grp in GROUPS:
        s = e_scr[grp[0]]
        for k in grp[1:]:
            s = s + e_scr[k]
        sums.append(s)
    tot = sums[0]
    for s in sums[1:]:
        tot = tot + s
    inv = 1.0 / tot
    for ch in range(6):
        o_ref[0, ch] = sums[ch] * inv


def _softmax_call(seg):
    return pl.pallas_call(
        _softmax_body,
        grid=(5, H // BH),
        in_specs=[pl.BlockSpec((1, 19, BH, W), lambda c, h: (c, 0, h, 0))],
        out_specs=pl.BlockSpec((1, 6, BH, W), lambda c, h: (c, 0, h, 0)),
        out_shape=jax.ShapeDtypeStruct((5, 6, H, W), jnp.float32),
        scratch_shapes=[pltpu.VMEM((19, BH, W), jnp.float32)],
    )(seg)


def _proj_body(pts_ref, trs_ref, r0_ref, ps_ref, pix_ref, w_ref):
    pts = pts_ref[...]
    r0m = r0_ref[...]
    rid = lax.broadcasted_iota(jnp.int32, (4, BN), 0)
    us, vs, zs, zcs = [], [], [], []
    for c in range(5):
        cam = jnp.dot(trs_ref[c], pts)
        cam3 = jnp.where(rid == 3, 1.0, cam)
        rc = jnp.dot(r0m, cam3)
        q = jnp.dot(ps_ref[c], rc)
        us.append(q[0:1])
        vs.append(q[1:2])
        zs.append(q[2:3])
        zcs.append(cam[2:3])
    U = jnp.concatenate(us, 0)
    V = jnp.concatenate(vs, 0)
    Q = jnp.concatenate(zs, 0)
    ZC = jnp.concatenate(zcs, 0)
    u = U / Q
    v = V / Q
    M = (u > 0.0) & (u < float(W)) & (v > 0.0) & (v < float(H)) & (ZC > 0.0)
    pxi = jnp.minimum(jnp.maximum(jnp.floor(u).astype(jnp.int32), 0), W - 1)
    pyi = jnp.minimum(jnp.maximum(jnp.floor(v).astype(jnp.int32), 0), H - 1)
    pix = pyi * W + pxi
    pid = pl.program_id(0)
    n_id = (pid * BN
            + lax.broadcasted_iota(jnp.int32, (5, BN), 1))
    pix = jnp.where(M, pix, n_id)
    cam_off = lax.broadcasted_iota(jnp.int32, (5, BN), 0) * (6 * HW)
    pix_ref[0:5, :] = pix + cam_off

    scale = jnp.full((1, BN), 1.0, jnp.float32)
    for a, b in ((0, 1), (0, 2), (1, 3), (2, 4)):
        scale = scale * jnp.where(M[a:a + 1] & M[b:b + 1], 0.5, 1.0)
    w_ref[0:5, :] = jnp.where(M, scale, 0.0)


def _proj_call(pts, trs, r0m, psm):
    return pl.pallas_call(
        _proj_body,
        grid=(NP // BN,),
        in_specs=[
            pl.BlockSpec((4, BN), lambda i: (0, i)),
            pl.BlockSpec((5, 4, 4), lambda i: (0, 0, 0)),
            pl.BlockSpec((4, 4), lambda i: (0, 0)),
            pl.BlockSpec((5, 3, 4), lambda i: (0, 0, 0)),
        ],
        out_specs=[
            pl.BlockSpec((8, BN), lambda i: (0, i)),
            pl.BlockSpec((8, BN), lambda i: (0, i)),
        ],
        out_shape=[
            jax.ShapeDtypeStruct((8, NP), jnp.int32),
            jax.ShapeDtypeStruct((8, NP), jnp.float32),
        ],
    )(pts, trs, r0m, psm)


def _paint_body(table_hbm, pix_hbm, w_hbm, out_hbm, pbuf, gbuf, wbuf, obuf,
                sem):
    wid = lax.axis_index("s") * 2 + lax.axis_index("c")

    def chunk_body(t, carry):
        n = wid * CPW + t
        cps = []
        for c in range(5):
            cps.append(pltpu.async_copy(
                pix_hbm.at[c, n], pbuf.at[pl.ds(c * CHUNK, CHUNK)], sem))
            cps.append(pltpu.async_copy(
                w_hbm.at[c, n], wbuf.at[pl.ds(c * CHUNK, CHUNK)], sem))
        for cp in cps:
            cp.wait()
        gps = []
        for ch in range(6):
            gps.append(pltpu.async_copy(
                table_hbm.at[pl.ds(ch * HW, 25 * HW)].at[pbuf],
                gbuf.at[pl.ds(ch * (5 * CHUNK), 5 * CHUNK)], sem))
        for gp in gps:
            gp.wait()

        def group_body(g, carry2):
            pb = g * 16
            wv = [wbuf[pl.ds(c * CHUNK + pb, 16)] for c in range(5)]
            for ch in range(6):
                base = ch * (5 * CHUNK) + pb
                acc = wv[0] * gbuf[pl.ds(base, 16)]
                for c in range(1, 5):
                    acc = acc + wv[c] * gbuf[pl.ds(base + c * CHUNK, 16)]
                obuf[ch, pl.ds(pb, 16)] = acc
            return carry2

        lax.fori_loop(0, CHUNK // 16, group_body, 0)
        pltpu.async_copy(
            obuf, out_hbm.at[:, pl.ds(n * CHUNK, CHUNK)], sem).wait()
        return carry

    lax.fori_loop(0, CPW, chunk_body, 0)


_paint_call = functools.partial(
    pl.kernel,
    out_type=jax.ShapeDtypeStruct((8, NP), jnp.float32),
    mesh=plsc.VectorSubcoreMesh(core_axis_name="c", subcore_axis_name="s",
                                num_cores=2, num_subcores=16),
    scratch_types=[
        pltpu.VMEM((5 * CHUNK,), jnp.int32),
        pltpu.VMEM((6 * 5 * CHUNK,), jnp.float32),
        pltpu.VMEM((5 * CHUNK,), jnp.float32),
        pltpu.VMEM((8, CHUNK), jnp.float32),
        pltpu.SemaphoreType.DMA,
    ],
)(_paint_body)


def kernel(segmentation_results, lidar_raw, Tr_velo_to_cam_0,
           Tr_velo_to_cam_1, Tr_velo_to_cam_2, Tr_velo_to_cam_3,
           Tr_velo_to_cam_4, P0, P1, P2, P3, P4, R0_rect):
    table = _softmax_call(segmentation_results)

    trs = jnp.stack([Tr_velo_to_cam_0, Tr_velo_to_cam_1, Tr_velo_to_cam_2,
                     Tr_velo_to_cam_3, Tr_velo_to_cam_4])
    psm = jnp.stack([P0, P1, P2, P3, P4])

    xyz = jnp.transpose(lidar_raw[:, :3])
    pts = jnp.concatenate(
        [xyz, jnp.ones((1, NPTS), jnp.float32)], 0)
    pts = jnp.pad(pts, ((0, 0), (0, NP - NPTS)))
    pix, w5 = _proj_call(pts, trs, R0_rect, psm)

    scores = _paint_call(
        table.reshape(5 * 6 * HW),
        pix.reshape(8, NCH, CHUNK),
        w5.reshape(8, NCH, CHUNK))

    return jnp.concatenate(
        [lidar_raw[:, :5], jnp.transpose(scores[:6, :NPTS])], axis=1)


# speedup vs baseline: 7.1359x; 1.2058x over previous
import functools

import jax
import jax.numpy as jnp
from jax import lax
from jax.experimental import pallas as pl
from jax.experimental.pallas import tpu as pltpu
from jax.experimental.pallas import tpu_sc as plsc

H = 384
W = 1024
HW = H * W
NPTS = 200000

NW = 32
CHUNK = 1024
CPW = 7
NP = NW * CPW * CHUNK
NCH = NP // CHUNK

GROUPS = [list(range(11)), [18], [13, 14, 15, 16], [11], [12], [17]]

BH = 8
BN = 8192


def _softmax_body(x_ref, o_ref, e_scr):
    for k in range(19):
        e_scr[k] = jnp.exp(x_ref[0, k])
    sums = []
    for grp in GROUPS:
        s = e_scr[grp[0]]
        for k in grp[1:]:
            s = s + e_scr[k]
        sums.append(s)
    tot = sums[0]
    for s in sums[1:]:
        tot = tot + s
    inv = 1.0 / tot
    for ch in range(6):
        o_ref[0, ch] = sums[ch] * inv


def _softmax_call(seg):
    return pl.pallas_call(
        _softmax_body,
        grid=(5, H // BH),
        in_specs=[pl.BlockSpec((1, 19, BH, W), lambda c, h: (c, 0, h, 0))],
        out_specs=pl.BlockSpec((1, 6, BH, W), lambda c, h: (c, 0, h, 0)),
        out_shape=jax.ShapeDtypeStruct((5, 6, H, W), jnp.float32),
        scratch_shapes=[pltpu.VMEM((19, BH, W), jnp.float32)],
    )(seg)


def _proj_body(pts_ref, trs_ref, r0_ref, ps_ref, pix_ref, w_ref):
    pts = pts_ref[...]
    r0m = r0_ref[...]
    rid = lax.broadcasted_iota(jnp.int32, (4, BN), 0)
    us, vs, zs, zcs = [], [], [], []
    for c in range(5):
        cam = jnp.dot(trs_ref[c], pts)
        cam3 = jnp.where(rid == 3, 1.0, cam)
        rc = jnp.dot(r0m, cam3)
        q = jnp.dot(ps_ref[c], rc)
        us.append(q[0:1])
        vs.append(q[1:2])
        zs.append(q[2:3])
        zcs.append(cam[2:3])
    U = jnp.concatenate(us, 0)
    V = jnp.concatenate(vs, 0)
    Q = jnp.concatenate(zs, 0)
    ZC = jnp.concatenate(zcs, 0)
    u = U / Q
    v = V / Q
    M = (u > 0.0) & (u < float(W)) & (v > 0.0) & (v < float(H)) & (ZC > 0.0)
    pxi = jnp.minimum(jnp.maximum(jnp.floor(u).astype(jnp.int32), 0), W - 1)
    pyi = jnp.minimum(jnp.maximum(jnp.floor(v).astype(jnp.int32), 0), H - 1)
    pix = pyi * W + pxi
    cam_off = lax.broadcasted_iota(jnp.int32, (5, BN), 0) * (6 * HW)
    pix_ref[0:5, :] = jnp.where(M, pix + cam_off, -1)

    scale = jnp.full((1, BN), 1.0, jnp.float32)
    for a, b in ((0, 1), (0, 2), (1, 3), (2, 4)):
        scale = scale * jnp.where(M[a:a + 1] & M[b:b + 1], 0.5, 1.0)
    w_ref[0:5, :] = jnp.where(M, scale, 0.0)


def _proj_call(pts, trs, r0m, psm):
    return pl.pallas_call(
        _proj_body,
        grid=(NP // BN,),
        in_specs=[
            pl.BlockSpec((4, BN), lambda i: (0, i)),
            pl.BlockSpec((5, 4, 4), lambda i: (0, 0, 0)),
            pl.BlockSpec((4, 4), lambda i: (0, 0)),
            pl.BlockSpec((5, 3, 4), lambda i: (0, 0, 0)),
        ],
        out_specs=[
            pl.BlockSpec((8, BN), lambda i: (0, i)),
            pl.BlockSpec((8, BN), lambda i: (0, i)),
        ],
        out_shape=[
            jax.ShapeDtypeStruct((8, NP), jnp.int32),
            jax.ShapeDtypeStruct((8, NP), jnp.float32),
        ],
    )(pts, trs, r0m, psm)


def _paint_body(table_hbm, pix_hbm, w_hbm, out_hbm, pbuf, gbuf, wbuf, obuf,
                sem):
    wid = lax.axis_index("s") * 2 + lax.axis_index("c")
    zero16 = jnp.zeros((16,), jnp.float32)

    def zero_body(i, carry0):
        gbuf[pl.ds(i * 16, 16)] = zero16
        return carry0

    lax.fori_loop(0, (6 * 5 * CHUNK) // 16, zero_body, 0)

    def chunk_body(t, carry):
        n = wid * CPW + t
        cps = []
        for c in range(5):
            cps.append(pltpu.async_copy(
                pix_hbm.at[c, n], pbuf.at[pl.ds(c * CHUNK, CHUNK)], sem))
            cps.append(pltpu.async_copy(
                w_hbm.at[c, n], wbuf.at[pl.ds(c * CHUNK, CHUNK)], sem))
        for cp in cps:
            cp.wait()
        gps = []
        for ch in range(6):
            gps.append(pltpu.async_copy(
                table_hbm.at[pl.ds(ch * HW, 25 * HW)].at[
                    plsc.Indices(pbuf, ignored_value=-1)],
                gbuf.at[pl.ds(ch * (5 * CHUNK), 5 * CHUNK)], sem))
        for gp in gps:
            gp.wait()

        def group_body(g, carry2):
            pb = g * 16
            wv = [wbuf[pl.ds(c * CHUNK + pb, 16)] for c in range(5)]
            for ch in range(6):
                base = ch * (5 * CHUNK) + pb
                acc = wv[0] * gbuf[pl.ds(base, 16)]
                for c in range(1, 5):
                    acc = acc + wv[c] * gbuf[pl.ds(base + c * CHUNK, 16)]
                obuf[ch, pl.ds(pb, 16)] = acc
            return carry2

        lax.fori_loop(0, CHUNK // 16, group_body, 0)
        pltpu.async_copy(
            obuf, out_hbm.at[:, pl.ds(n * CHUNK, CHUNK)], sem).wait()
        return carry

    lax.fori_loop(0, CPW, chunk_body, 0)


_paint_call = functools.partial(
    pl.kernel,
    out_type=jax.ShapeDtypeStruct((8, NP), jnp.float32),
    mesh=plsc.VectorSubcoreMesh(core_axis_name="c", subcore_axis_name="s",
                                num_cores=2, num_subcores=16),
    scratch_types=[
        pltpu.VMEM((5 * CHUNK,), jnp.int32),
        pltpu.VMEM((6 * 5 * CHUNK,), jnp.float32),
        pltpu.VMEM((5 * CHUNK,), jnp.float32),
        pltpu.VMEM((8, CHUNK), jnp.float32),
        pltpu.SemaphoreType.DMA,
    ],
)(_paint_body)


def kernel(segmentation_results, lidar_raw, Tr_velo_to_cam_0,
           Tr_velo_to_cam_1, Tr_velo_to_cam_2, Tr_velo_to_cam_3,
           Tr_velo_to_cam_4, P0, P1, P2, P3, P4, R0_rect):
    table = _softmax_call(segmentation_results)

    trs = jnp.stack([Tr_velo_to_cam_0, Tr_velo_to_cam_1, Tr_velo_to_cam_2,
                     Tr_velo_to_cam_3, Tr_velo_to_cam_4])
    psm = jnp.stack([P0, P1, P2, P3, P4])

    xyz = jnp.transpose(lidar_raw[:, :3])
    pts = jnp.concatenate(
        [xyz, jnp.ones((1, NPTS), jnp.float32)], 0)
    pts = jnp.pad(pts, ((0, 0), (0, NP - NPTS)))
    pix, w5 = _proj_call(pts, trs, R0_rect, psm)

    scores = _paint_call(
        table.reshape(5 * 6 * HW),
        pix.reshape(8, NCH, CHUNK),
        w5.reshape(8, NCH, CHUNK))

    return jnp.concatenate(
        [lidar_raw[:, :5], jnp.transpose(scores[:6, :NPTS])], axis=1)


# speedup vs baseline: 7.5857x; 1.0630x over previous
import functools

import jax
import jax.numpy as jnp
from jax import lax
from jax.experimental import pallas as pl
from jax.experimental.pallas import tpu as pltpu
from jax.experimental.pallas import tpu_sc as plsc

H = 384
W = 1024
HW = H * W
NPTS = 200000

NW = 32
CHUNK = 1024
CPW = 7
NP = NW * CPW * CHUNK
NCH = NP // CHUNK

GROUPS = [list(range(11)), [18], [13, 14, 15, 16], [11], [12], [17]]

BH = 8
BN = 8192


def _softmax_body(x_ref, o_ref, e_scr):
    for k in range(19):
        e_scr[k] = jnp.exp(x_ref[0, k])
    sums = []
    for grp in GROUPS:
        s = e_scr[grp[0]]
        for k in grp[1:]:
            s = s + e_scr[k]
        sums.append(s)
    tot = sums[0]
    for s in sums[1:]:
        tot = tot + s
    inv = 1.0 / tot
    for ch in range(6):
        o_ref[0, ch] = sums[ch] * inv


def _softmax_call(seg):
    return pl.pallas_call(
        _softmax_body,
        grid=(5, H // BH),
        in_specs=[pl.BlockSpec((1, 19, BH, W), lambda c, h: (c, 0, h, 0))],
        out_specs=pl.BlockSpec((1, 6, BH, W), lambda c, h: (c, 0, h, 0)),
        out_shape=jax.ShapeDtypeStruct((5, 6, H, W), jnp.float32),
        scratch_shapes=[pltpu.VMEM((19, BH, W), jnp.float32)],
    )(seg)


def _proj_body(pts_ref, trs_ref, r0_ref, ps_ref, pix_ref, w_ref):
    pts = pts_ref[...]
    r0m = r0_ref[...]
    rid = lax.broadcasted_iota(jnp.int32, (4, BN), 0)
    us, vs, zs, zcs = [], [], [], []
    for c in range(5):
        cam = jnp.dot(trs_ref[c], pts)
        cam3 = jnp.where(rid == 3, 1.0, cam)
        rc = jnp.dot(r0m, cam3)
        q = jnp.dot(ps_ref[c], rc)
        us.append(q[0:1])
        vs.append(q[1:2])
        zs.append(q[2:3])
        zcs.append(cam[2:3])
    U = jnp.concatenate(us, 0)
    V = jnp.concatenate(vs, 0)
    Q = jnp.concatenate(zs, 0)
    ZC = jnp.concatenate(zcs, 0)
    u = U / Q
    v = V / Q
    M = (u > 0.0) & (u < float(W)) & (v > 0.0) & (v < float(H)) & (ZC > 0.0)
    pxi = jnp.minimum(jnp.maximum(jnp.floor(u).astype(jnp.int32), 0), W - 1)
    pyi = jnp.minimum(jnp.maximum(jnp.floor(v).astype(jnp.int32), 0), H - 1)
    pix = pyi * W + pxi
    cam_off = lax.broadcasted_iota(jnp.int32, (5, BN), 0) * (6 * HW)
    pix_ref[0:5, :] = jnp.where(M, pix + cam_off, -1)

    scale = jnp.full((1, BN), 1.0, jnp.float32)
    for a, b in ((0, 1), (0, 2), (1, 3), (2, 4)):
        scale = scale * jnp.where(M[a:a + 1] & M[b:b + 1], 0.5, 1.0)
    w_ref[0:5, :] = jnp.where(M, scale, 0.0)


def _proj_call(pts, trs, r0m, psm):
    return pl.pallas_call(
        _proj_body,
        grid=(NP // BN,),
        in_specs=[
            pl.BlockSpec((4, BN), lambda i: (0, i)),
            pl.BlockSpec((5, 4, 4), lambda i: (0, 0, 0)),
            pl.BlockSpec((4, 4), lambda i: (0, 0)),
            pl.BlockSpec((5, 3, 4), lambda i: (0, 0, 0)),
        ],
        out_specs=[
            pl.BlockSpec((8, BN), lambda i: (0, i)),
            pl.BlockSpec((8, BN), lambda i: (0, i)),
        ],
        out_shape=[
            jax.ShapeDtypeStruct((8, NP), jnp.int32),
            jax.ShapeDtypeStruct((8, NP), jnp.float32),
        ],
    )(pts, trs, r0m, psm)


_SC_MESH = plsc.VectorSubcoreMesh(core_axis_name="c", subcore_axis_name="s",
                                  num_cores=2, num_subcores=16)


def _paint_body(table_hbm, pix_hbm, w_hbm, zinit_hbm, out_hbm, pbufs, gbufs,
                wbufs, obuf, lsem, gsem, osem):
    wid = lax.axis_index("s") * 2 + lax.axis_index("c")
    row16 = lax.iota(jnp.int32, 16)
    pltpu.sync_copy(zinit_hbm, gbufs[0])
    pltpu.sync_copy(zinit_hbm, gbufs[1])

    def issue_loads(t, b):
        n = wid * CPW + t
        cps = []
        for c in range(5):
            cps.append(pltpu.async_copy(
                pix_hbm.at[c, n], pbufs[b].at[pl.ds(c * CHUNK, CHUNK)],
                lsem))
            cps.append(pltpu.async_copy(
                w_hbm.at[c, n], wbufs[b].at[pl.ds(c * CHUNK, CHUNK)], lsem))
        return cps

    def issue_gathers(b):
        return [pltpu.async_copy(
            table_hbm.at[pl.ds(ch * HW, 25 * HW)].at[
                plsc.Indices(pbufs[b], ignored_value=-1)],
            gbufs[b].at[pl.ds(ch * (5 * CHUNK), 5 * CHUNK)], gsem)
            for ch in range(6)]

    def combine(t, b):
        n = wid * CPW + t

        def group_body(g, carry2):
            pb = g * 16
            wv = [wbufs[b][pl.ds(c * CHUNK + pb, 16)] for c in range(5)]
            for ch in range(6):
                base = ch * (5 * CHUNK) + pb
                acc = wv[0] * gbufs[b][pl.ds(base, 16)]
                for c in range(1, 5):
                    acc = acc + wv[c] * gbufs[b][
                        pl.ds(base + c * CHUNK, 16)]
                obuf[ch, pl.ds(pb, 16)] = acc
            return carry2

        lax.fori_loop(0, CHUNK // 16, group_body, 0)
        pltpu.async_copy(
            obuf, out_hbm.at[:, pl.ds(n * CHUNK, CHUNK)], osem).wait()

    lds = issue_loads(0, 0)
    for cp in lds:
        cp.wait()
    gth = issue_gathers(0)
    for t in range(CPW):
        b = t % 2
        nb = 1 - b
        if t + 1 < CPW:
            nlds = issue_loads(t + 1, nb)
        for gp in gth:
            gp.wait()
        if t + 1 < CPW:
            for cp in nlds:
                cp.wait()
            gth = issue_gathers(nb)
        combine(t, b)


_paint_call = functools.partial(
    pl.kernel,
    out_type=jax.ShapeDtypeStruct((8, NP), jnp.float32),
    mesh=_SC_MESH,
    compiler_params=pltpu.CompilerParams(needs_layout_passes=False),
    scratch_types=[
        [pltpu.VMEM((5 * CHUNK,), jnp.int32)] * 2,
        [pltpu.VMEM((6 * 5 * CHUNK,), jnp.float32)] * 2,
        [pltpu.VMEM((5 * CHUNK,), jnp.float32)] * 2,
        pltpu.VMEM((8, CHUNK), jnp.float32),
        pltpu.SemaphoreType.DMA,
        pltpu.SemaphoreType.DMA,
        pltpu.SemaphoreType.DMA,
    ],
)(_paint_body)


def kernel(segmentation_results, lidar_raw, Tr_velo_to_cam_0,
           Tr_velo_to_cam_1, Tr_velo_to_cam_2, Tr_velo_to_cam_3,
           Tr_velo_to_cam_4, P0, P1, P2, P3, P4, R0_rect):
    table = _softmax_call(segmentation_results)

    trs = jnp.stack([Tr_velo_to_cam_0, Tr_velo_to_cam_1, Tr_velo_to_cam_2,
                     Tr_velo_to_cam_3, Tr_velo_to_cam_4])
    psm = jnp.stack([P0, P1, P2, P3, P4])

    xyz = jnp.transpose(lidar_raw[:, :3])
    pts = jnp.concatenate(
        [xyz, jnp.ones((1, NPTS), jnp.float32)], 0)
    pts = jnp.pad(pts, ((0, 0), (0, NP - NPTS)))
    pix, w5 = _proj_call(pts, trs, R0_rect, psm)

    scores = _paint_call(
        table.reshape(5 * 6 * HW),
        pix.reshape(8, NCH, CHUNK),
        w5.reshape(8, NCH, CHUNK),
        jnp.zeros((6 * 5 * CHUNK,), jnp.float32))

    return jnp.concatenate(
        [lidar_raw[:, :5], jnp.transpose(scores[:6, :NPTS])], axis=1)


# speedup vs baseline: 7.6022x; 1.0022x over previous
import functools

import jax
import jax.numpy as jnp
from jax import lax
from jax.experimental import pallas as pl
from jax.experimental.pallas import tpu as pltpu
from jax.experimental.pallas import tpu_sc as plsc

H = 384
W = 1024
HW = H * W
NPTS = 200000

NW = 32
CHUNK = 1024
CPW = 7
NP = NW * CPW * CHUNK
NCH = NP // CHUNK

GROUPS = [list(range(11)), [18], [13, 14, 15, 16], [11], [12], [17]]

BH = 8
BN = 8192


def _softmax_body(x_ref, o_ref, e_scr):
    for k in range(19):
        e_scr[k] = jnp.exp(x_ref[0, k])
    sums = []
    for grp in GROUPS:
        s = e_scr[grp[0]]
        for k in grp[1:]:
            s = s + e_scr[k]
        sums.append(s)
    tot = sums[0]
    for s in sums[1:]:
        tot = tot + s
    inv = 1.0 / tot
    for ch in range(6):
        o_ref[0, ch] = sums[ch] * inv


def _softmax_call(seg):
    return pl.pallas_call(
        _softmax_body,
        grid=(5, H // BH),
        in_specs=[pl.BlockSpec((1, 19, BH, W), lambda c, h: (c, 0, h, 0))],
        out_specs=pl.BlockSpec((1, 6, BH, W), lambda c, h: (c, 0, h, 0)),
        out_shape=jax.ShapeDtypeStruct((5, 6, H, W), jnp.float32),
        scratch_shapes=[pltpu.VMEM((19, BH, W), jnp.float32)],
    )(seg)


def _proj_body(pts_ref, trs_ref, r0_ref, ps_ref, pix_ref, w_ref):
    pts = pts_ref[...]
    r0m = r0_ref[...]
    rid = lax.broadcasted_iota(jnp.int32, (4, BN), 0)
    us, vs, zs, zcs = [], [], [], []
    for c in range(5):
        cam = jnp.dot(trs_ref[c], pts)
        cam3 = jnp.where(rid == 3, 1.0, cam)
        rc = jnp.dot(r0m, cam3)
        q = jnp.dot(ps_ref[c], rc)
        us.append(q[0:1])
        vs.append(q[1:2])
        zs.append(q[2:3])
        zcs.append(cam[2:3])
    U = jnp.concatenate(us, 0)
    V = jnp.concatenate(vs, 0)
    Q = jnp.concatenate(zs, 0)
    ZC = jnp.concatenate(zcs, 0)
    u = U / Q
    v = V / Q
    M = (u > 0.0) & (u < float(W)) & (v > 0.0) & (v < float(H)) & (ZC > 0.0)
    pxi = jnp.minimum(jnp.maximum(jnp.floor(u).astype(jnp.int32), 0), W - 1)
    pyi = jnp.minimum(jnp.maximum(jnp.floor(v).astype(jnp.int32), 0), H - 1)
    pix = pyi * W + pxi
    cam_off = lax.broadcasted_iota(jnp.int32, (5, BN), 0) * (6 * HW)
    pix_ref[0:5, :] = jnp.where(M, pix + cam_off, -1)

    scale = jnp.full((1, BN), 1.0, jnp.float32)
    for a, b in ((0, 1), (0, 2), (1, 3), (2, 4)):
        scale = scale * jnp.where(M[a:a + 1] & M[b:b + 1], 0.5, 1.0)
    w_ref[0:5, :] = jnp.where(M, scale, 0.0)


def _proj_call(pts, trs, r0m, psm):
    return pl.pallas_call(
        _proj_body,
        grid=(NP // BN,),
        in_specs=[
            pl.BlockSpec((4, BN), lambda i: (0, i)),
            pl.BlockSpec((5, 4, 4), lambda i: (0, 0, 0)),
            pl.BlockSpec((4, 4), lambda i: (0, 0)),
            pl.BlockSpec((5, 3, 4), lambda i: (0, 0, 0)),
        ],
        out_specs=[
            pl.BlockSpec((8, BN), lambda i: (0, i)),
            pl.BlockSpec((8, BN), lambda i: (0, i)),
        ],
        out_shape=[
            jax.ShapeDtypeStruct((8, NP), jnp.int32),
            jax.ShapeDtypeStruct((8, NP), jnp.float32),
        ],
    )(pts, trs, r0m, psm)


_SC_MESH = plsc.VectorSubcoreMesh(core_axis_name="c", subcore_axis_name="s",
                                  num_cores=2, num_subcores=16)


def _paint_body(table_hbm, pix_hbm, w_hbm, zinit_hbm, out_hbm, pbufs, gbufs,
                wbufs, obuf, lsem, gsem, osem):
    wid = lax.axis_index("s") * 2 + lax.axis_index("c")
    row16 = lax.iota(jnp.int32, 16)
    pltpu.sync_copy(zinit_hbm, gbufs[0])
    pltpu.sync_copy(zinit_hbm, gbufs[1])

    def issue_loads(t, b):
        n = wid * CPW + t
        cps = []
        for c in range(5):
            cps.append(pltpu.async_copy(
                pix_hbm.at[c, n], pbufs[b].at[pl.ds(c * CHUNK, CHUNK)],
                lsem))
            cps.append(pltpu.async_copy(
                w_hbm.at[c, n], wbufs[b].at[pl.ds(c * CHUNK, CHUNK)], lsem))
        return cps

    def issue_gathers(b):
        return [pltpu.async_copy(
            table_hbm.at[pl.ds(ch * HW, 25 * HW)].at[
                plsc.Indices(pbufs[b], ignored_value=-1)],
            gbufs[b].at[pl.ds(ch * (5 * CHUNK), 5 * CHUNK)], gsem)
            for ch in range(6)]

    def combine(t, b):
        n = wid * CPW + t

        def group_body(g, carry2):
            pb = g * 16
            wv = [wbufs[b][pl.ds(c * CHUNK + pb, 16)] for c in range(5)]
            for ch in range(6):
                base = ch * (5 * CHUNK) + pb
                acc = wv[0] * gbufs[b][pl.ds(base, 16)]
                for c in range(1, 5):
                    acc = acc + wv[c] * gbufs[b][
                        pl.ds(base + c * CHUNK, 16)]
                obuf[ch, pl.ds(pb, 16)] = acc
            return carry2

        lax.fori_loop(0, CHUNK // 16, group_body, 0)
        pltpu.async_copy(
            obuf, out_hbm.at[:, pl.ds(n * CHUNK, CHUNK)], osem).wait()

    lds = issue_loads(0, 0)
    for cp in lds:
        cp.wait()
    gth = issue_gathers(0)
    for t in range(CPW):
        b = t % 2
        nb = 1 - b
        if t + 1 < CPW:
            nlds = issue_loads(t + 1, nb)
        for gp in gth:
            gp.wait()
        if t + 1 < CPW:
            for cp in nlds:
                cp.wait()
            gth = issue_gathers(nb)
        combine(t, b)


_paint_call = functools.partial(
    pl.kernel,
    out_type=jax.ShapeDtypeStruct((8, NP), jnp.float32),
    mesh=_SC_MESH,
    compiler_params=pltpu.CompilerParams(needs_layout_passes=False, use_tc_tiling_on_sc=True),
    scratch_types=[
        [pltpu.VMEM((5 * CHUNK,), jnp.int32)] * 2,
        [pltpu.VMEM((6 * 5 * CHUNK,), jnp.float32)] * 2,
        [pltpu.VMEM((5 * CHUNK,), jnp.float32)] * 2,
        pltpu.VMEM((8, CHUNK), jnp.float32),
        pltpu.SemaphoreType.DMA,
        pltpu.SemaphoreType.DMA,
        pltpu.SemaphoreType.DMA,
    ],
)(_paint_body)


def kernel(segmentation_results, lidar_raw, Tr_velo_to_cam_0,
           Tr_velo_to_cam_1, Tr_velo_to_cam_2, Tr_velo_to_cam_3,
           Tr_velo_to_cam_4, P0, P1, P2, P3, P4, R0_rect):
    table = _softmax_call(segmentation_results)

    trs = jnp.stack([Tr_velo_to_cam_0, Tr_velo_to_cam_1, Tr_velo_to_cam_2,
                     Tr_velo_to_cam_3, Tr_velo_to_cam_4])
    psm = jnp.stack([P0, P1, P2, P3, P4])

    xyz = jnp.transpose(lidar_raw[:, :3])
    pts = jnp.concatenate(
        [xyz, jnp.ones((1, NPTS), jnp.float32)], 0)
    pts = jnp.pad(pts, ((0, 0), (0, NP - NPTS)))
    pix, w5 = _proj_call(pts, trs, R0_rect, psm)

    scores = _paint_call(
        table.reshape(5 * 6 * HW),
        pix.reshape(8, NCH, CHUNK),
        w5.reshape(8, NCH, CHUNK),
        jnp.zeros((6 * 5 * CHUNK,), jnp.float32))

    return jnp.concatenate(
        [lidar_raw[:, :5], jnp.transpose(scores[:6, :NPTS])], axis=1)
